# baseline (speedup 1.0000x reference)
import functools
import jax
import jax.numpy as jnp
from jax import lax
from jax.experimental import pallas as pl
from jax.experimental.pallas import tpu as pltpu
from jax.experimental.pallas import tpu_sc as plsc

N = 10000
D = 128
E = 320000
K = 5
NC = 2
NS = 16
NW = NC * NS
CH = 128
CPT = 80
E2 = NW * CPT * CH
EROWS = E2 // CH
VROWS = E // CH
NP = 10240
SP = NP // NS
RPT = EROWS // NS


def _rsqrt16(p):
    i = lax.bitcast_convert_type(p, jnp.int32)
    i = jnp.int32(0x5F3759DF) - lax.shift_right_logical(i, 1)
    y = lax.bitcast_convert_type(i, jnp.float32)
    for _ in range(4):
        y = y * (1.5 - 0.5 * p * y * y)
    return y


def _lane_shuffle(v, idx):
    return lax.gather(
        v, idx[:, None],
        dimension_numbers=lax.GatherDimensionNumbers(
            offset_dims=(), collapsed_slice_dims=(0,), start_index_map=(0,)),
        slice_sizes=(1,),
        mode=lax.GatherScatterMode.PROMISE_IN_BOUNDS)


def _tanh16(t):
    a = jnp.abs(t)
    e = jnp.exp(-2.0 * a)
    return jnp.sign(t) * ((1.0 - e) / (1.0 + e))


def _tc_dense(x, wt, b):
    bn = 400

    def body(x_ref, w_ref, b_ref, h_ref, n_ref):
        acc = jnp.dot(x_ref[...], w_ref[...],
                      preferred_element_type=jnp.float32)
        hb = jnp.tanh(acc + b_ref[...])
        h_ref[...] = hb
        n_ref[...] = jnp.sqrt(jnp.sum(hb * hb, axis=1))[None, None, :]

    return pl.pallas_call(
        body,
        grid=(N // bn,),
        in_specs=[
            pl.BlockSpec((bn, D), lambda i: (i, 0)),
            pl.BlockSpec((D, D), lambda i: (0, 0)),
            pl.BlockSpec((1, D), lambda i: (0, 0)),
        ],
        out_specs=[
            pl.BlockSpec((bn, D), lambda i: (i, 0)),
            pl.BlockSpec((1, 1, bn), lambda i: (i, 0, 0)),
        ],
        out_shape=[
            jax.ShapeDtypeStruct((N, D), jnp.float32),
            jax.ShapeDtypeStruct((N // bn, 1, bn), jnp.float32),
        ],
    )(x, wt, b)


def _sc_edge_weights(h, nrm, srcp, dstp):
    mesh = plsc.VectorSubcoreMesh(core_axis_name="c", subcore_axis_name="s")

    @functools.partial(
        pl.kernel,
        out_type=jax.ShapeDtypeStruct((EROWS, CH), jnp.float32),
        mesh=mesh,
        compiler_params=pltpu.CompilerParams(needs_layout_passes=False),
        scratch_types=[
            pltpu.VMEM((2, CH), jnp.int32),
            pltpu.VMEM((2, CH), jnp.int32),
            pltpu.VMEM((2, CH, D), jnp.float32),
            pltpu.VMEM((2, CH, D), jnp.float32),
            pltpu.VMEM((2, CH), jnp.float32),
            pltpu.VMEM((N,), jnp.float32),
            pltpu.SemaphoreType.DMA,
            pltpu.SemaphoreType.DMA,
        ],
    )
    def kern(h_hbm, nrm_hbm, src_hbm, dst_hbm, ew_hbm,
             idx_s, idx_d, rows_s, rows_d, ew_v, nv, sem0, sem1):
        wid = lax.axis_index("s") * NC + lax.axis_index("c")
        row0 = wid * CPT
        iota = lax.iota(jnp.int32, 16)
        sems = [sem0, sem1]

        pltpu.sync_copy(nrm_hbm, nv)

        def fire(j, b):
            pltpu.sync_copy(src_hbm.at[row0 + j], idx_s.at[b])
            pltpu.sync_copy(dst_hbm.at[row0 + j], idx_d.at[b])
            pltpu.async_copy(h_hbm.at[idx_s.at[b]], rows_s.at[b], sems[b])
            pltpu.async_copy(h_hbm.at[idx_d.at[b]], rows_d.at[b], sems[b])

        fire(jnp.int32(0), 0)
        fire(jnp.int32(1), 1)

        def process(j, b):
            pltpu.make_async_copy(h_hbm.at[idx_s.at[b]], rows_s.at[b],
                                  sems[b]).wait()
            pltpu.make_async_copy(h_hbm.at[idx_d.at[b]], rows_d.at[b],
                                  sems[b]).wait()
            validf = jnp.where(row0 + j < VROWS, 1.0, 0.0).astype(jnp.float32)

            @plsc.parallel_loop(0, CH // 16, unroll=2)
            def group(g):
                num = jnp.zeros((16,), jnp.float32)
                for e in range(16):
                    ge = g * 16 + e
                    a0 = jnp.zeros((16,), jnp.float32)
                    a1 = jnp.zeros((16,), jnp.float32)
                    for u in range(D // 32):
                        sv0 = rows_s[b, ge, pl.ds(u * 32, 16)]
                        dv0 = rows_d[b, ge, pl.ds(u * 32, 16)]
                        sv1 = rows_s[b, ge, pl.ds(u * 32 + 16, 16)]
                        dv1 = rows_d[b, ge, pl.ds(u * 32 + 16, 16)]
                        a0 = a0 + sv0 * dv0
                        a1 = a1 + sv1 * dv1
                    s = a0 + a1
                    for sh in (8, 4, 2, 1):
                        s = s + _lane_shuffle(s, jnp.bitwise_xor(iota, sh))
                    onehot = jnp.where(iota == e, 1.0, 0.0)
                    num = num + s * onehot
                si = idx_s[b, pl.ds(g * 16, 16)]
                di = idx_d[b, pl.ds(g * 16, 16)]
                ns = plsc.load_gather(nv, [si])
                nd = plsc.load_gather(nv, [di])
                den = jnp.maximum(ns * nd, 1e-8)
                w = jnp.maximum(num / den, 0.0) * validf
                ew_v[b, pl.ds(g * 16, 16)] = w
            pltpu.sync_copy(ew_v.at[b], ew_hbm.at[row0 + j])

            @pl.when(j + 2 < CPT)
            def _():
                fire(j + 2, b)

        def pair(ci, carry):
            process(2 * ci, 0)
            process(2 * ci + 1, 1)
            return carry

        lax.fori_loop(0, CPT // 2, pair, 0)

    return kern(h, nrm, srcp, dstp)


def _sc_appnp(srcp, dstp, ew2d, mpad, cst):
    mesh = plsc.VectorSubcoreMesh(core_axis_name="c", subcore_axis_name="s")

    @functools.partial(
        pl.kernel,
        out_type=jax.ShapeDtypeStruct((NP,), jnp.float32),
        mesh=mesh,
        compiler_params=pltpu.CompilerParams(needs_layout_passes=False),
        scratch_types=[
            pltpu.VMEM((RPT, CH), jnp.int32),
            pltpu.VMEM((RPT, CH), jnp.int32),
            pltpu.VMEM((RPT, CH), jnp.float32),
            pltpu.VMEM((RPT, CH), jnp.float32),
            pltpu.VMEM((NP,), jnp.float32),
            pltpu.VMEM((NP,), jnp.float32),
            pltpu.VMEM((NP,), jnp.float32),
            pltpu.VMEM((SP,), jnp.float32),
            pltpu.VMEM((SP,), jnp.float32),
            pltpu.VMEM((16,), jnp.float32),
            pltpu.VMEM_SHARED((NP,), jnp.float32),
            pltpu.VMEM_SHARED((NP,), jnp.float32),
            pltpu.SemaphoreType.DMA,
        ],
    )
    def kern(src_hbm, dst_hbm, ew_hbm, m_hbm, cst_hbm, out_hbm,
             src_v, dst_v, msg_v, nrm_v, dis_v, out_v, h0_v,
             sl_v, selfw_v, cst_v, acc_sp, dis_sp, sem):
        sid = lax.axis_index("s")
        cid = lax.axis_index("c")
        iota = lax.iota(jnp.int32, 16)
        r0 = sid * RPT
        n0 = sid * SP

        pltpu.sync_copy(cst_hbm, cst_v)
        cvec = cst_v[...]
        alpha = cvec[0]
        sp = cvec[1]

        pltpu.sync_copy(src_hbm.at[pl.ds(r0, RPT)], src_v)
        pltpu.sync_copy(dst_hbm.at[pl.ds(r0, RPT)], dst_v)
        pltpu.sync_copy(ew_hbm.at[pl.ds(r0, RPT)], msg_v)
        pltpu.sync_copy(m_hbm, h0_v)

        def relu_body(i, carry):
            v = jnp.maximum(h0_v[pl.ds(i * 16, 16)], 0.0)
            h0_v[pl.ds(i * 16, 16)] = v
            out_v[pl.ds(i * 16, 16)] = v
            return carry
        lax.fori_loop(0, NP // 16, relu_body, 0)

        def ones_body(i, carry):
            sl_v[pl.ds(i * 16, 16)] = jnp.full((16,), 1.0, jnp.float32)
            return carry
        lax.fori_loop(0, SP // 16, ones_body, 0)
        pltpu.sync_copy(sl_v, acc_sp.at[pl.ds(n0, SP)])
        plsc.subcore_barrier()

        def deg_fire(j, carry):
            pltpu.async_copy(msg_v.at[j], acc_sp.at[dst_v.at[j]], sem,
                             add=True)
            return carry
        lax.fori_loop(0, RPT, deg_fire, 0)
        pltpu.make_async_copy(ew_hbm.at[pl.ds(r0, RPT)], msg_v, sem).wait()
        plsc.subcore_barrier()

        pltpu.sync_copy(acc_sp.at[pl.ds(n0, SP)], sl_v)

        def dis_body(i, carry):
            dg = sl_v[pl.ds(i * 16, 16)]
            selfw_v[pl.ds(i * 16, 16)] = 1.0 / dg
            sl_v[pl.ds(i * 16, 16)] = _rsqrt16(dg)
            return carry
        lax.fori_loop(0, SP // 16, dis_body, 0)
        pltpu.sync_copy(sl_v, dis_sp.at[pl.ds(n0, SP)])
        plsc.subcore_barrier()
        pltpu.sync_copy(dis_sp, dis_v)

        def nrm_body(j, carry):
            for c in range(CH // 16):
                si = src_v[j, pl.ds(c * 16, 16)]
                di = dst_v[j, pl.ds(c * 16, 16)]
                dsv = plsc.load_gather(dis_v, [si])
                ddv = plsc.load_gather(dis_v, [di])
                w = msg_v[j, pl.ds(c * 16, 16)]
                nrm_v[j, pl.ds(c * 16, 16)] = dsv * w * ddv
            return carry
        lax.fori_loop(0, RPT, nrm_body, 0)

        def appnp_body(_it, carry):
            def init_body(i, c2):
                t = selfw_v[pl.ds(i * 16, 16)] * out_v[pl.ds(n0 + i * 16, 16)]
                sl_v[pl.ds(i * 16, 16)] = t
                return c2
            lax.fori_loop(0, SP // 16, init_body, 0)
            pltpu.sync_copy(sl_v, acc_sp.at[pl.ds(n0, SP)])
            plsc.subcore_barrier()

            def msg_body(j, c2):
                for c in range(CH // 16):
                    si = src_v[j, pl.ds(c * 16, 16)]
                    ov = plsc.load_gather(out_v, [si])
                    msg_v[j, pl.ds(c * 16, 16)] = \
                        ov * nrm_v[j, pl.ds(c * 16, 16)]
                return c2
            lax.fori_loop(0, RPT, msg_body, 0)

            def sc_fire(j, c2):
                pltpu.async_copy(msg_v.at[j], acc_sp.at[dst_v.at[j]], sem,
                                 add=True)
                return c2
            lax.fori_loop(0, RPT, sc_fire, 0)
            pltpu.make_async_copy(ew_hbm.at[pl.ds(r0, RPT)], msg_v,
                                  sem).wait()
            plsc.subcore_barrier()

            pltpu.sync_copy(acc_sp, out_v)

            def upd_body(i, c2):
                o = out_v[pl.ds(i * 16, 16)]
                h = h0_v[pl.ds(i * 16, 16)]
                out_v[pl.ds(i * 16, 16)] = (1.0 - alpha) * o + alpha * h
                return c2
            lax.fori_loop(0, NP // 16, upd_body, 0)
            plsc.subcore_barrier()
            return carry
        lax.fori_loop(0, K, appnp_body, 0)

        def fin_body(i, carry):
            t = out_v[pl.ds(n0 + i * 16, 16)] - sp
            sl_v[pl.ds(i * 16, 16)] = _tanh16(t)
            return carry
        lax.fori_loop(0, SP // 16, fin_body, 0)

        @pl.when(cid == 0)
        def _():
            pltpu.sync_copy(sl_v, out_hbm.at[pl.ds(n0, SP)])

    return kern(srcp, dstp, ew2d, mpad, cst)


def kernel(x, mask, edge_index, W, b_lin, alpha, bias):
    h, nrm3 = _tc_dense(x, W.T, b_lin.reshape(1, D))
    nrm = nrm3.reshape(N)

    pad = jnp.zeros((2, E2 - E), jnp.int32)
    eip = jnp.concatenate([edge_index.astype(jnp.int32), pad], axis=1)
    srcp = eip[0].reshape(EROWS, CH)
    dstp = eip[1].reshape(EROWS, CH)

    ew2d = _sc_edge_weights(h, nrm, srcp, dstp)

    mpad = jnp.zeros((NP,), jnp.float32).at[:N].set(mask[:, 0])
    sp = jax.nn.softplus(bias)[0]
    cst = jnp.zeros((16,), jnp.float32).at[0].set(alpha).at[1].set(sp)

    out_pad = _sc_appnp(srcp, dstp, ew2d, mpad, cst)

    out = out_pad[:N].reshape(N, 1)
    ew = ew2d.reshape(-1)[:E]
    return (out, ew)


# speedup vs baseline: 1.0720x; 1.0720x over previous
import functools
import jax
import jax.numpy as jnp
from jax import lax
from jax.experimental import pallas as pl
from jax.experimental.pallas import tpu as pltpu
from jax.experimental.pallas import tpu_sc as plsc

N = 10000
D = 128
E = 320000
K = 5
NC = 2
NS = 16
NW = NC * NS
CH = 128
CPT = 80
CPT0 = 106
CPT1 = 2 * CPT - CPT0
E2 = NW * CPT * CH
EROWS = E2 // CH
VROWS = E // CH
NP = 10240
SP = NP // NS
RPT = EROWS // NS


def _rsqrt16(p):
    i = lax.bitcast_convert_type(p, jnp.int32)
    i = jnp.int32(0x5F3759DF) - lax.shift_right_logical(i, 1)
    y = lax.bitcast_convert_type(i, jnp.float32)
    for _ in range(4):
        y = y * (1.5 - 0.5 * p * y * y)
    return y


def _tanh16(t):
    a = jnp.abs(t)
    e = jnp.exp(-2.0 * a)
    return jnp.sign(t) * ((1.0 - e) / (1.0 + e))


def _tc_dense(x, wt, b):
    bn = 400

    def body(x_ref, w_ref, b_ref, h_ref, n_ref):
        acc = jnp.dot(x_ref[...], w_ref[...],
                      preferred_element_type=jnp.float32)
        hb = jnp.tanh(acc + b_ref[...])
        h_ref[...] = hb
        n_ref[...] = jnp.sqrt(jnp.sum(hb * hb, axis=1))[None, None, :]

    return pl.pallas_call(
        body,
        grid=(N // bn,),
        in_specs=[
            pl.BlockSpec((bn, D), lambda i: (i, 0)),
            pl.BlockSpec((D, D), lambda i: (0, 0)),
            pl.BlockSpec((1, D), lambda i: (0, 0)),
        ],
        out_specs=[
            pl.BlockSpec((bn, D), lambda i: (i, 0)),
            pl.BlockSpec((1, 1, bn), lambda i: (i, 0, 0)),
        ],
        out_shape=[
            jax.ShapeDtypeStruct((N, D), jnp.float32),
            jax.ShapeDtypeStruct((N // bn, 1, bn), jnp.float32),
        ],
    )(x, wt, b)


def _sc_edge_weights(h, nrm, srcp, dstp):
    mesh = plsc.VectorSubcoreMesh(core_axis_name="c", subcore_axis_name="s")

    @functools.partial(
        pl.kernel,
        out_type=jax.ShapeDtypeStruct((EROWS, CH), jnp.float32),
        mesh=mesh,
        compiler_params=pltpu.CompilerParams(needs_layout_passes=False),
        scratch_types=[
            pltpu.VMEM((2, CH), jnp.int32),
            pltpu.VMEM((2, CH), jnp.int32),
            pltpu.VMEM((2, CH, D), jnp.float32),
            pltpu.VMEM((2, CH, D), jnp.float32),
            pltpu.VMEM((2, CH), jnp.float32),
            pltpu.VMEM((N,), jnp.float32),
            pltpu.SemaphoreType.DMA,
            pltpu.SemaphoreType.DMA,
        ],
    )
    def kern(h_hbm, nrm_hbm, src_hbm, dst_hbm, ew_hbm,
             idx_s, idx_d, rows_s, rows_d, ew_v, nv, sem0, sem1):
        sid = lax.axis_index("s")
        cid = lax.axis_index("c")
        row0 = jnp.where(cid == 0, sid * CPT0, NS * CPT0 + sid * CPT1)
        cpt = jnp.where(cid == 0, CPT0, CPT1)
        iota = lax.iota(jnp.int32, 16)
        sems = [sem0, sem1]

        pltpu.sync_copy(nrm_hbm, nv)

        def fire(j, b):
            pltpu.sync_copy(src_hbm.at[row0 + j], idx_s.at[b])
            pltpu.sync_copy(dst_hbm.at[row0 + j], idx_d.at[b])
            pltpu.async_copy(h_hbm.at[idx_s.at[b]], rows_s.at[b], sems[b])
            pltpu.async_copy(h_hbm.at[idx_d.at[b]], rows_d.at[b], sems[b])

        fire(jnp.int32(0), 0)
        fire(jnp.int32(1), 1)

        def process(j, b):
            pltpu.make_async_copy(h_hbm.at[idx_s.at[b]], rows_s.at[b],
                                  sems[b]).wait()
            pltpu.make_async_copy(h_hbm.at[idx_d.at[b]], rows_d.at[b],
                                  sems[b]).wait()
            validf = jnp.where(row0 + j < VROWS, 1.0, 0.0).astype(jnp.float32)

            def group(g, carry):
                num = jnp.zeros((16,), jnp.float32)
                for e in range(16):
                    ge = g * 16 + e
                    a0 = jnp.zeros((16,), jnp.float32)
                    a1 = jnp.zeros((16,), jnp.float32)
                    for u in range(D // 32):
                        sv0 = rows_s[b, ge, pl.ds(u * 32, 16)]
                        dv0 = rows_d[b, ge, pl.ds(u * 32, 16)]
                        sv1 = rows_s[b, ge, pl.ds(u * 32 + 16, 16)]
                        dv1 = rows_d[b, ge, pl.ds(u * 32 + 16, 16)]
                        a0 = a0 + sv0 * dv0
                        a1 = a1 + sv1 * dv1
                    s = jnp.sum(a0 + a1)
                    onehot = jnp.where(iota == e, 1.0, 0.0)
                    num = num + s * onehot
                si = idx_s[b, pl.ds(g * 16, 16)]
                di = idx_d[b, pl.ds(g * 16, 16)]
                ns = plsc.load_gather(nv, [si])
                nd = plsc.load_gather(nv, [di])
                den = jnp.maximum(ns * nd, 1e-8)
                w = jnp.maximum(num / den, 0.0) * validf
                ew_v[b, pl.ds(g * 16, 16)] = w
                return carry

            lax.fori_loop(0, CH // 16, group, 0)
            pltpu.sync_copy(ew_v.at[b], ew_hbm.at[row0 + j])

            @pl.when(j + 2 < cpt)
            def _():
                fire(j + 2, b)

        def pair(ci, carry):
            process(2 * ci, 0)
            process(2 * ci + 1, 1)
            return carry

        lax.fori_loop(0, cpt // 2, pair, 0)

    return kern(h, nrm, srcp, dstp)


def _sc_appnp(srcp, dstp, ew2d, mpad, cst):
    mesh = plsc.VectorSubcoreMesh(core_axis_name="c", subcore_axis_name="s")

    @functools.partial(
        pl.kernel,
        out_type=jax.ShapeDtypeStruct((NP,), jnp.float32),
        mesh=mesh,
        compiler_params=pltpu.CompilerParams(needs_layout_passes=False),
        scratch_types=[
            pltpu.VMEM((RPT, CH), jnp.int32),
            pltpu.VMEM((RPT, CH), jnp.int32),
            pltpu.VMEM((RPT, CH), jnp.float32),
            pltpu.VMEM((RPT, CH), jnp.float32),
            pltpu.VMEM((NP,), jnp.float32),
            pltpu.VMEM((NP,), jnp.float32),
            pltpu.VMEM((NP,), jnp.float32),
            pltpu.VMEM((SP,), jnp.float32),
            pltpu.VMEM((SP,), jnp.float32),
            pltpu.VMEM((16,), jnp.float32),
            pltpu.VMEM_SHARED((NP,), jnp.float32),
            pltpu.VMEM_SHARED((NP,), jnp.float32),
            pltpu.SemaphoreType.DMA,
        ],
    )
    def kern(src_hbm, dst_hbm, ew_hbm, m_hbm, cst_hbm, out_hbm,
             src_v, dst_v, msg_v, nrm_v, dis_v, out_v, h0_v,
             sl_v, selfw_v, cst_v, acc_sp, dis_sp, sem):
        sid = lax.axis_index("s")
        cid = lax.axis_index("c")
        iota = lax.iota(jnp.int32, 16)
        r0 = sid * RPT
        n0 = sid * SP

        pltpu.sync_copy(cst_hbm, cst_v)
        cvec = cst_v[...]
        alpha = cvec[0]
        sp = cvec[1]

        pltpu.sync_copy(src_hbm.at[pl.ds(r0, RPT)], src_v)
        pltpu.sync_copy(dst_hbm.at[pl.ds(r0, RPT)], dst_v)
        pltpu.sync_copy(ew_hbm.at[pl.ds(r0, RPT)], msg_v)
        pltpu.sync_copy(m_hbm, h0_v)

        def relu_body(i, carry):
            v = jnp.maximum(h0_v[pl.ds(i * 16, 16)], 0.0)
            h0_v[pl.ds(i * 16, 16)] = v
            out_v[pl.ds(i * 16, 16)] = v
            return carry
        lax.fori_loop(0, NP // 16, relu_body, 0)

        def ones_body(i, carry):
            sl_v[pl.ds(i * 16, 16)] = jnp.full((16,), 1.0, jnp.float32)
            return carry
        lax.fori_loop(0, SP // 16, ones_body, 0)
        pltpu.sync_copy(sl_v, acc_sp.at[pl.ds(n0, SP)])
        plsc.subcore_barrier()

        def deg_fire(j, carry):
            pltpu.async_copy(msg_v.at[j], acc_sp.at[dst_v.at[j]], sem,
                             add=True)
            return carry
        lax.fori_loop(0, RPT, deg_fire, 0)
        pltpu.make_async_copy(ew_hbm.at[pl.ds(r0, RPT)], msg_v, sem).wait()
        plsc.subcore_barrier()

        pltpu.sync_copy(acc_sp.at[pl.ds(n0, SP)], sl_v)

        def dis_body(i, carry):
            dg = sl_v[pl.ds(i * 16, 16)]
            selfw_v[pl.ds(i * 16, 16)] = 1.0 / dg
            sl_v[pl.ds(i * 16, 16)] = _rsqrt16(dg)
            return carry
        lax.fori_loop(0, SP // 16, dis_body, 0)
        pltpu.sync_copy(sl_v, dis_sp.at[pl.ds(n0, SP)])
        plsc.subcore_barrier()
        pltpu.sync_copy(dis_sp, dis_v)

        def nrm_body(j, carry):
            for c in range(CH // 16):
                si = src_v[j, pl.ds(c * 16, 16)]
                di = dst_v[j, pl.ds(c * 16, 16)]
                dsv = plsc.load_gather(dis_v, [si])
                ddv = plsc.load_gather(dis_v, [di])
                w = msg_v[j, pl.ds(c * 16, 16)]
                nrm_v[j, pl.ds(c * 16, 16)] = dsv * w * ddv
            return carry
        lax.fori_loop(0, RPT, nrm_body, 0)

        def appnp_body(_it, carry):
            def init_body(i, c2):
                t = selfw_v[pl.ds(i * 16, 16)] * out_v[pl.ds(n0 + i * 16, 16)]
                sl_v[pl.ds(i * 16, 16)] = t
                return c2
            lax.fori_loop(0, SP // 16, init_body, 0)
            pltpu.sync_copy(sl_v, acc_sp.at[pl.ds(n0, SP)])
            plsc.subcore_barrier()

            def msg_body(j, c2):
                for c in range(CH // 16):
                    si = src_v[j, pl.ds(c * 16, 16)]
                    ov = plsc.load_gather(out_v, [si])
                    msg_v[j, pl.ds(c * 16, 16)] = \
                        ov * nrm_v[j, pl.ds(c * 16, 16)]
                return c2
            lax.fori_loop(0, RPT, msg_body, 0)

            def sc_fire(j, c2):
                pltpu.async_copy(msg_v.at[j], acc_sp.at[dst_v.at[j]], sem,
                                 add=True)
                return c2
            lax.fori_loop(0, RPT, sc_fire, 0)
            pltpu.make_async_copy(ew_hbm.at[pl.ds(r0, RPT)], msg_v,
                                  sem).wait()
            plsc.subcore_barrier()

            pltpu.sync_copy(acc_sp, out_v)

            def upd_body(i, c2):
                o = out_v[pl.ds(i * 16, 16)]
                h = h0_v[pl.ds(i * 16, 16)]
                out_v[pl.ds(i * 16, 16)] = (1.0 - alpha) * o + alpha * h
                return c2
            lax.fori_loop(0, NP // 16, upd_body, 0)
            plsc.subcore_barrier()
            return carry
        lax.fori_loop(0, K, appnp_body, 0)

        def fin_body(i, carry):
            t = out_v[pl.ds(n0 + i * 16, 16)] - sp
            sl_v[pl.ds(i * 16, 16)] = _tanh16(t)
            return carry
        lax.fori_loop(0, SP // 16, fin_body, 0)

        @pl.when(cid == 0)
        def _():
            pltpu.sync_copy(sl_v, out_hbm.at[pl.ds(n0, SP)])

    return kern(srcp, dstp, ew2d, mpad, cst)


def kernel(x, mask, edge_index, W, b_lin, alpha, bias):
    h, nrm3 = _tc_dense(x, W.T, b_lin.reshape(1, D))
    nrm = nrm3.reshape(N)

    pad = jnp.zeros((2, E2 - E), jnp.int32)
    eip = jnp.concatenate([edge_index.astype(jnp.int32), pad], axis=1)
    srcp = eip[0].reshape(EROWS, CH)
    dstp = eip[1].reshape(EROWS, CH)

    ew2d = _sc_edge_weights(h, nrm, srcp, dstp)

    mpad = jnp.zeros((NP,), jnp.float32).at[:N].set(mask[:, 0])
    sp = jax.nn.softplus(bias)[0]
    cst = jnp.zeros((16,), jnp.float32).at[0].set(alpha).at[1].set(sp)

    out_pad = _sc_appnp(srcp, dstp, ew2d, mpad, cst)

    out = out_pad[:N].reshape(N, 1)
    ew = ew2d.reshape(-1)[:E]
    return (out, ew)
